# baseline (speedup 1.0000x reference)
import functools

import jax
import jax.numpy as jnp
from jax import lax
from jax.experimental import pallas as pl
from jax.experimental.pallas import tpu as pltpu
from jax.experimental.pallas import tpu_sc as plsc

ENT_N = 14541
REL_N = 237
DIM = 64
BATCH = 16384

NC = 2
NS = 16
NW = NC * NS
BPW = BATCH // NW
CHUNK = 128
NCHUNK = BPW // CHUNK
RTAB = 480
RCHUNK = RTAB // 4
ENT_PAIRS = (ENT_N + 1) // 2
REL_PAIRS = (REL_N * (DIM + 1) + 1) // 2

_INV1024 = 1.0 / 1024.0
_FEED = (0, 8, 4, 12, 2, 10, 6, 14, 1, 9, 5, 13, 3, 11, 7, 15)

_GDN = lax.GatherDimensionNumbers(
    offset_dims=(), collapsed_slice_dims=(0,), start_index_map=(0,))


def _take16(v, idx):
    return lax.gather(v, idx[:, None], _GDN, (1,),
                      mode=lax.GatherScatterMode.PROMISE_IN_BOUNDS)


def _tree_sum(vecs, perms, iota16):
    cur = list(vecs)
    for k, lanebit in ((3, 8), (2, 4), (1, 2), (0, 1)):
        sel = (iota16 & lanebit) == 0
        nxt = []
        for i in range(0, len(cur), 2):
            a, b = cur[i], cur[i + 1]
            sa = a + _take16(a, perms[k])
            sb = b + _take16(b, perms[k])
            nxt.append(jnp.where(sel, sa, sb))
        cur = nxt
    return cur[0]


def _rsqrt_nr(m):
    bits = lax.bitcast_convert_type(m, jnp.int32)
    y = lax.bitcast_convert_type(
        jnp.int32(0x5F3759DF) - lax.shift_right_logical(bits, 1),
        jnp.float32)
    for _ in range(3):
        y = y * (1.5 - 0.5 * m * y * y)
    return y


def _row_vecs(j, scale, rrow, rh, oh, hh, th, reltab_v, hbuf_v, tbuf_v,
              row0):
    num = jnp.zeros((16,), jnp.float32)
    p2 = jnp.zeros((16,), jnp.float32)
    t2 = jnp.zeros((16,), jnp.float32)
    for s in range(2):
        hv = plsc.unpack(hbuf_v[row0 + j, pl.ds(hh + s * 32, 32)],
                         format=plsc.PackFormat.INTERLEAVED)
        tv = plsc.unpack(tbuf_v[row0 + j, pl.ds(th + s * 32, 32)],
                         format=plsc.PackFormat.INTERLEAVED)
        rv = plsc.unpack(reltab_v[rrow, pl.ds(rh + s * 32, 32)],
                         format=plsc.PackFormat.INTERLEAVED)
        ov = plsc.unpack(reltab_v[rrow + 1, pl.ds(oh + s * 32, 32)],
                         format=plsc.PackFormat.INTERLEAVED)
        for q in range(2):
            offs = scale * ov[q]
            x = hv[q] + rv[q] - offs
            prod = x * x * _INV1024 + offs
            num = num + prod * tv[q]
            p2 = p2 + prod * prod
            t2 = t2 + tv[q] * tv[q]
    return num, p2, t2


def _build_sc_kernel():
    mesh = plsc.VectorSubcoreMesh(core_axis_name="c", subcore_axis_name="s")

    @functools.partial(
        pl.kernel,
        mesh=mesh,
        out_type=jax.ShapeDtypeStruct((BATCH,), jnp.float32),
        compiler_params=pltpu.CompilerParams(
            use_tc_tiling_on_sc=False, needs_layout_passes=False),
        scratch_types=[
            pltpu.VMEM((4, RCHUNK), jnp.int32),
            pltpu.VMEM((RTAB, 2 * DIM), jnp.bfloat16),
            pltpu.VMEM((BPW,), jnp.int32),
            pltpu.VMEM((BPW,), jnp.int32),
            pltpu.VMEM((BPW,), jnp.int32),
            pltpu.VMEM((BPW,), jnp.int32),
            pltpu.VMEM((BPW, 2 * DIM), jnp.bfloat16),
            pltpu.VMEM((BPW, 2 * DIM), jnp.bfloat16),
            pltpu.VMEM((BPW,), jnp.int32),
            pltpu.VMEM((BPW,), jnp.float32),
            pltpu.SemaphoreType.DMA,
        ],
    )
    def sc_kernel(ent_hbm, rel_hbm, hpair_hbm, tpair_hbm, hoff_hbm,
                  toff_hbm, rt_hbm, ridx_hbm, out_hbm, relidx_v, reltab_v,
                  hpair_v, tpair_v, hoff_v, toff_v, hbuf_v, tbuf_v, rt_v,
                  out_v, sem0):
        wid = lax.axis_index("s") * NC + lax.axis_index("c")
        base = wid * BPW
        bsl = pl.ds(base, BPW)

        stage = [
            pltpu.async_copy(rt_hbm.at[bsl], rt_v, sem0),
            pltpu.async_copy(ridx_hbm, relidx_v, sem0),
            pltpu.async_copy(hpair_hbm.at[bsl], hpair_v, sem0),
            pltpu.async_copy(tpair_hbm.at[bsl], tpair_v, sem0),
            pltpu.async_copy(hoff_hbm.at[bsl], hoff_v, sem0),
            pltpu.async_copy(toff_hbm.at[bsl], toff_v, sem0),
        ]
        for cp in stage:
            cp.wait()

        gathers = []
        for i in range(4):
            gathers.append(pltpu.async_copy(
                rel_hbm.at[relidx_v.at[i]],
                reltab_v.at[pl.ds(i * RCHUNK, RCHUNK)], sem0))
        for c in range(NCHUNK):
            sl = pl.ds(c * CHUNK, CHUNK)
            gathers.append(pltpu.async_copy(
                ent_hbm.at[hpair_v.at[sl]], hbuf_v.at[sl], sem0))
            gathers.append(pltpu.async_copy(
                ent_hbm.at[tpair_v.at[sl]], tbuf_v.at[sl], sem0))
        for cp in gathers:
            cp.wait()

        iota16 = lax.iota(jnp.int32, 16)
        perms = [iota16 ^ (1 << k) for k in range(4)]

        def body(grp, carry):
            row0 = grp * 16
            gsl = pl.ds(row0, 16)
            rt16 = rt_v[gsl]
            scale16 = (rt16 + 1).astype(jnp.float32)
            rh16 = (rt16 & 1) * 64
            oh16 = 64 - rh16
            hh16 = hoff_v[gsl]
            th16 = toff_v[gsl]
            nvecs, pvecs, tvecs = [], [], []
            for j in _FEED:
                num, p2, t2 = _row_vecs(
                    j, scale16[j], rt16[j] * 2, rh16[j], oh16[j], hh16[j],
                    th16[j], reltab_v, hbuf_v, tbuf_v, row0)
                nvecs.append(num)
                pvecs.append(p2)
                tvecs.append(t2)
            nsum = _tree_sum(nvecs, perms, iota16)
            psum = _tree_sum(pvecs, perms, iota16)
            tsum = _tree_sum(tvecs, perms, iota16)
            m = jnp.maximum(psum, 1e-16) * jnp.maximum(tsum, 1e-16)
            out_v[gsl] = -(nsum * _rsqrt_nr(m))
            return carry

        lax.fori_loop(0, BPW // 16, body, 0)
        pltpu.sync_copy(out_v, out_hbm.at[bsl])

    return sc_kernel


_SC_KERNEL = _build_sc_kernel()


def kernel(h_ids, r_typ, t_ids, ent_emb, rel_emb):
    i = jnp.arange(RTAB, dtype=jnp.int32)
    rrows = jnp.minimum(i >> 1, REL_N - 1) * (DIM + 1) + (i & 1)
    ridx = (rrows >> 1).reshape(4, RCHUNK)
    h_ids = h_ids.astype(jnp.int32)
    t_ids = t_ids.astype(jnp.int32)
    ent16 = jnp.pad(ent_emb.astype(jnp.bfloat16), ((0, 1), (0, 0)))
    rel16 = jnp.pad(rel_emb.astype(jnp.bfloat16), ((0, 1), (0, 0)))
    return _SC_KERNEL(ent16.reshape(ENT_PAIRS, 2 * DIM),
                      rel16.reshape(REL_PAIRS, 2 * DIM),
                      h_ids >> 1, t_ids >> 1,
                      (h_ids & 1) << 6, (t_ids & 1) << 6,
                      r_typ.astype(jnp.int32), ridx)


# speedup vs baseline: 1.4172x; 1.4172x over previous
import functools

import jax
import jax.numpy as jnp
from jax import lax
from jax.experimental import pallas as pl
from jax.experimental.pallas import tpu as pltpu
from jax.experimental.pallas import tpu_sc as plsc

ENT_N = 14541
REL_N = 237
DIM = 64
BATCH = 16384

NC = 2
NS = 16
NW = NC * NS
BPW = BATCH // NW
CHUNK = 128
NCHUNK = BPW // CHUNK
RTAB = 2 * REL_N

_INV1024 = 1.0 / 1024.0
_FEED = (0, 8, 4, 12, 2, 10, 6, 14, 1, 9, 5, 13, 3, 11, 7, 15)

_GDN = lax.GatherDimensionNumbers(
    offset_dims=(), collapsed_slice_dims=(0,), start_index_map=(0,))


def _take16(v, idx):
    return lax.gather(v, idx[:, None], _GDN, (1,),
                      mode=lax.GatherScatterMode.PROMISE_IN_BOUNDS)


def _tree_sum(vecs, perms, iota16):
    cur = list(vecs)
    for k, lanebit in ((3, 8), (2, 4), (1, 2), (0, 1)):
        sel = (iota16 & lanebit) == 0
        nxt = []
        for i in range(0, len(cur), 2):
            a, b = cur[i], cur[i + 1]
            sa = a + _take16(a, perms[k])
            sb = b + _take16(b, perms[k])
            nxt.append(jnp.where(sel, sa, sb))
        cur = nxt
    return cur[0]


def _rsqrt_nr(m):
    bits = lax.bitcast_convert_type(m, jnp.int32)
    y = lax.bitcast_convert_type(
        jnp.int32(0x5F3759DF) - lax.shift_right_logical(bits, 1),
        jnp.float32)
    for _ in range(3):
        y = y * (1.5 - 0.5 * m * y * y)
    return y


def _row_vecs(j, scale, rrow, reltab_v, hbuf_v, tbuf_v, row0):
    num = jnp.zeros((16,), jnp.float32)
    p2 = jnp.zeros((16,), jnp.float32)
    t2 = jnp.zeros((16,), jnp.float32)
    for s in range(DIM // 16):
        sl = pl.ds(s * 16, 16)
        hv = hbuf_v[row0 + j, sl]
        tv = tbuf_v[row0 + j, sl]
        rv = reltab_v[rrow, sl]
        ov = reltab_v[rrow + 1, sl]
        offs = scale * ov
        x = hv + rv - offs
        prod = x * x * _INV1024 + offs
        num = num + prod * tv
        p2 = p2 + prod * prod
        t2 = t2 + tv * tv
    return num, p2, t2


def _build_sc_kernel():
    mesh = plsc.VectorSubcoreMesh(core_axis_name="c", subcore_axis_name="s")

    @functools.partial(
        pl.kernel,
        mesh=mesh,
        out_type=jax.ShapeDtypeStruct((BATCH,), jnp.float32),
        compiler_params=pltpu.CompilerParams(
            use_tc_tiling_on_sc=False, needs_layout_passes=False),
        scratch_types=[
            pltpu.VMEM((RTAB, DIM), jnp.float32),
            pltpu.VMEM((BPW,), jnp.int32),
            pltpu.VMEM((BPW,), jnp.int32),
            pltpu.VMEM((BPW, DIM), jnp.float32),
            pltpu.VMEM((BPW, DIM), jnp.float32),
            pltpu.VMEM((BPW,), jnp.int32),
            pltpu.VMEM((BPW,), jnp.float32),
            pltpu.SemaphoreType.DMA,
        ],
    )
    def sc_kernel(ent_hbm, relsub_hbm, hid_hbm, tid_hbm, rt_hbm, out_hbm,
                  reltab_v, idxh_v, idxt_v, hbuf_v, tbuf_v, rt_v, out_v,
                  sem0):
        wid = lax.axis_index("s") * NC + lax.axis_index("c")
        base = wid * BPW
        bsl = pl.ds(base, BPW)

        stage = [
            pltpu.async_copy(relsub_hbm, reltab_v, sem0),
            pltpu.async_copy(rt_hbm.at[bsl], rt_v, sem0),
            pltpu.async_copy(hid_hbm.at[bsl], idxh_v, sem0),
            pltpu.async_copy(tid_hbm.at[bsl], idxt_v, sem0),
        ]
        for cp in stage:
            cp.wait()

        gathers = []
        for c in range(NCHUNK):
            sl = pl.ds(c * CHUNK, CHUNK)
            gathers.append(pltpu.async_copy(
                ent_hbm.at[idxh_v.at[sl]], hbuf_v.at[sl], sem0))
            gathers.append(pltpu.async_copy(
                ent_hbm.at[idxt_v.at[sl]], tbuf_v.at[sl], sem0))
        for cp in gathers:
            cp.wait()

        iota16 = lax.iota(jnp.int32, 16)
        perms = [iota16 ^ (1 << k) for k in range(4)]

        def body(grp, carry):
            row0 = grp * 16
            gsl = pl.ds(row0, 16)
            rt16 = rt_v[gsl]
            scale16 = (rt16 + 1).astype(jnp.float32)
            nvecs, pvecs, tvecs = [], [], []
            for j in _FEED:
                num, p2, t2 = _row_vecs(j, scale16[j], rt16[j] * 2,
                                        reltab_v, hbuf_v, tbuf_v, row0)
                nvecs.append(num)
                pvecs.append(p2)
                tvecs.append(t2)
            nsum = _tree_sum(nvecs, perms, iota16)
            psum = _tree_sum(pvecs, perms, iota16)
            tsum = _tree_sum(tvecs, perms, iota16)
            m = jnp.maximum(psum, 1e-16) * jnp.maximum(tsum, 1e-16)
            out_v[gsl] = -(nsum * _rsqrt_nr(m))
            return carry

        lax.fori_loop(0, BPW // 16, body, 0)
        pltpu.sync_copy(out_v, out_hbm.at[bsl])

    return sc_kernel


_SC_KERNEL = _build_sc_kernel()


def kernel(h_ids, r_typ, t_ids, ent_emb, rel_emb):
    i = jnp.arange(RTAB, dtype=jnp.int32)
    sub_rows = (i >> 1) * (DIM + 1) + (i & 1)
    rel_sub = jnp.take(rel_emb, sub_rows, axis=0)
    return _SC_KERNEL(ent_emb, rel_sub, h_ids.astype(jnp.int32),
                      t_ids.astype(jnp.int32), r_typ.astype(jnp.int32))
